# baseline (speedup 1.0000x reference)
import functools
import jax
import jax.numpy as jnp
from jax import lax
from jax.experimental import pallas as pl
from jax.experimental.pallas import tpu as pltpu, tpu_sc as plsc

N_ROWS = 100000
K = 32
FAT_COLS = 128
FAT = FAT_COLS // K

SC_SLAB_ROWS = 800
N_SC_SLABS = 32
SC_ROWS = SC_SLAB_ROWS * N_SC_SLABS
TC_ROWS = N_ROWS - SC_ROWS
TC_FAT = TC_ROWS // FAT
SC_FAT0 = TC_FAT
G_PER_SLAB = SC_SLAB_ROWS // 16
FAT_PER_SLAB = SC_SLAB_ROWS // FAT
CHUNKS = ((0, 104), (104, 96))
IDX_PAD = 208

TC_BLOCK_FAT = 600
TC_GRID = TC_FAT // TC_BLOCK_FAT


def _sc_body(neigh_hbm, consts_hbm, out_hbm, buf2d, outbuf, idxr, cbuf,
             sem, osem):
    wid = lax.axis_index("s") * 2 + lax.axis_index("c")
    pltpu.sync_copy(consts_hbm, cbuf)
    c1 = cbuf[pl.ds(0, 16)]
    wv = cbuf[pl.ds(16, 16)]
    bg = cbuf[pl.ds(32, 16)]
    wr = cbuf[pl.ds(48, 16)]
    br = cbuf[pl.ds(64, 16)]
    lane = lax.iota(jnp.int32, 16)
    lane4 = lax.shift_right_logical(lane, 2)
    lane3_32 = (lane & 3) * K

    base = SC_FAT0 + wid * FAT_PER_SLAB
    for t in range(IDX_PAD // 16):
        idxr[pl.ds(t * 16, 16)] = base + t * 16 + lane
    for off, sz in CHUNKS:
        pltpu.async_copy(
            neigh_hbm.at[idxr.at[pl.ds(off, sz)]],
            buf2d.at[pl.ds(off, sz), :],
            sem)
    for off, sz in CHUNKS:
        pltpu.make_async_copy(
            neigh_hbm.at[idxr.at[pl.ds(off, sz)]],
            buf2d.at[pl.ds(off, sz), :],
            sem).wait()

    def pairgroup(j, carry):
        fata = lane4 + j * 8
        fatb = fata + 4
        dena = [jnp.zeros((16,), jnp.float32) for _ in range(4)]
        svna = [jnp.zeros((16,), jnp.float32) for _ in range(4)]
        denb = [jnp.zeros((16,), jnp.float32) for _ in range(4)]
        svnb = [jnp.zeros((16,), jnp.float32) for _ in range(4)]
        for k in range(K):
            col = lane3_32 + ((lane + k) & (K - 1))
            va = plsc.load_gather(buf2d, [fata, col])
            vb = plsc.load_gather(buf2d, [fatb, col])
            ta = va * c1
            tb = vb * c1
            ea = jnp.exp(jnp.maximum(ta, ta * 0.2))
            eb = jnp.exp(jnp.maximum(tb, tb * 0.2))
            dena[k % 4] = dena[k % 4] + ea
            svna[k % 4] = svna[k % 4] + va * ea
            denb[k % 4] = denb[k % 4] + eb
            svnb[k % 4] = svnb[k % 4] + vb * eb
        da = (dena[0] + dena[1]) + (dena[2] + dena[3])
        sa = (svna[0] + svna[1]) + (svna[2] + svna[3])
        db = (denb[0] + denb[1]) + (denb[2] + denb[3])
        sb = (svnb[0] + svnb[1]) + (svnb[2] + svnb[3])
        oa = (sa * wv) / (da + 1e-16) + bg
        ob = (sb * wv) / (db + 1e-16) + bg
        ra = jnp.where(oa > 0, oa, jnp.exp(oa) - 1.0)
        rb = jnp.where(ob > 0, ob, jnp.exp(ob) - 1.0)
        outbuf[pl.ds(j * 32, 16)] = ra * wr + br
        outbuf[pl.ds(j * 32 + 16, 16)] = rb * wr + br
        return carry

    lax.fori_loop(0, G_PER_SLAB // 2, pairgroup, 0)

    pltpu.async_copy(
        outbuf, out_hbm.at[pl.ds(wid * SC_SLAB_ROWS, SC_SLAB_ROWS)], osem)
    pltpu.make_async_copy(
        outbuf, out_hbm.at[pl.ds(wid * SC_SLAB_ROWS, SC_SLAB_ROWS)],
        osem).wait()


def _tc_body(x_ref, c1s, wvs, bgs, wrs, brs, o_ref):
    x = x_ref[...]
    c1 = c1s[0, 0]
    t = x * c1
    e = jnp.exp(jnp.maximum(t, t * 0.2))
    seg = (lax.broadcasted_iota(jnp.int32, (FAT_COLS, FAT), 0) // K
           == lax.broadcasted_iota(jnp.int32, (FAT_COLS, FAT), 1)
           ).astype(jnp.float32)
    dn = (((1,), (0,)), ((), ()))
    den = lax.dot_general(e, seg, dn, preferred_element_type=jnp.float32)
    sv = lax.dot_general(x * e, seg, dn, preferred_element_type=jnp.float32)
    o = (sv * wvs[0, 0]) / (den + 1e-16) + bgs[0, 0]
    r = jnp.where(o > 0, o, jnp.exp(o) - 1.0)
    o_ref[...] = r * wrs[0, 0] + brs[0, 0]


def kernel(query_emb, entity_emb, neighbors, W, a_src, a_tgt, b_gat, W_rank, b_rank):
    n = neighbors.shape[0]
    neigh2d = neighbors.reshape(n * K // FAT_COLS, FAT_COLS)
    w0 = W[0, 0]
    c1 = w0 * a_src[0, 0, 0]
    consts = jnp.concatenate([
        jnp.full((16,), c1, jnp.float32),
        jnp.full((16,), w0, jnp.float32),
        jnp.full((16,), b_gat[0], jnp.float32),
        jnp.full((16,), W_rank[0, 0], jnp.float32),
        jnp.full((16,), b_rank[0], jnp.float32),
    ])

    mesh = plsc.VectorSubcoreMesh(core_axis_name="c", subcore_axis_name="s")
    sc_run = functools.partial(
        pl.kernel,
        mesh=mesh,
        compiler_params=pltpu.CompilerParams(needs_layout_passes=False),
        out_type=jax.ShapeDtypeStruct((SC_ROWS,), jnp.float32),
        scratch_types=[
            pltpu.VMEM((FAT_PER_SLAB, FAT_COLS), jnp.float32),
            pltpu.VMEM((SC_SLAB_ROWS,), jnp.float32),
            pltpu.VMEM((IDX_PAD,), jnp.int32),
            pltpu.VMEM((80,), jnp.float32),
            pltpu.SemaphoreType.DMA,
            pltpu.SemaphoreType.DMA,
        ],
    )(_sc_body)
    out_sc = sc_run(neigh2d, consts)

    s11 = lambda v: jnp.full((1, 1), v, jnp.float32)
    smem_spec = pl.BlockSpec(memory_space=pltpu.SMEM)
    out_tc = pl.pallas_call(
        _tc_body,
        grid=(TC_GRID,),
        in_specs=[
            pl.BlockSpec((TC_BLOCK_FAT, FAT_COLS), lambda i: (i, 0)),
            smem_spec, smem_spec, smem_spec, smem_spec, smem_spec,
        ],
        out_specs=pl.BlockSpec((TC_BLOCK_FAT, FAT), lambda i: (i, 0)),
        out_shape=jax.ShapeDtypeStruct((TC_FAT, FAT), jnp.float32),
        compiler_params=pltpu.CompilerParams(
            dimension_semantics=("parallel",)),
    )(neigh2d, s11(c1), s11(w0), s11(b_gat[0]), s11(W_rank[0, 0]),
      s11(b_rank[0]))

    out = jnp.concatenate([out_tc.reshape(TC_ROWS), out_sc])
    return out.reshape(n, 1)


# speedup vs baseline: 1.0617x; 1.0617x over previous
import functools
import jax
import jax.numpy as jnp
from jax import lax
from jax.experimental import pallas as pl
from jax.experimental.pallas import tpu as pltpu, tpu_sc as plsc

N_ROWS = 100000
K = 32
FAT_COLS = 128
FAT = FAT_COLS // K

SC_SLAB_ROWS = 800
N_SC_SLABS = 64
SC_ROWS = SC_SLAB_ROWS * N_SC_SLABS
TC_ROWS = N_ROWS - SC_ROWS
TC_FAT = TC_ROWS // FAT
SC_FAT0 = TC_FAT
G_PER_SLAB = SC_SLAB_ROWS // 16
FAT_PER_SLAB = SC_SLAB_ROWS // FAT
N_WORKERS = 32
SLABS_PER_W = N_SC_SLABS // N_WORKERS
CHUNKS = ((0, 104), (104, 96))
IDX_PAD = 208

TC_BLOCK_FAT = 488
TC_GRID = TC_FAT // TC_BLOCK_FAT


def _sc_body(neigh_hbm, consts_hbm, out_hbm, buf0, buf1, outbuf0, outbuf1,
             idx0, idx1, cbuf, sem0, sem1, osem0, osem1):
    wid = lax.axis_index("s") * 2 + lax.axis_index("c")
    pltpu.sync_copy(consts_hbm, cbuf)
    c1 = cbuf[pl.ds(0, 16)]
    wv = cbuf[pl.ds(16, 16)]
    bg = cbuf[pl.ds(32, 16)]
    wr = cbuf[pl.ds(48, 16)]
    br = cbuf[pl.ds(64, 16)]
    lane = lax.iota(jnp.int32, 16)
    lane4 = lax.shift_right_logical(lane, 2)
    lane3_32 = (lane & 3) * K

    sems = (sem0, sem1)
    bufs = (buf0, buf1)
    osems = (osem0, osem1)
    outbufs = (outbuf0, outbuf1)
    idxs = (idx0, idx1)

    def compute_slab(buf2d, outbuf):
        def pairgroup(j, carry):
            fata = lane4 + j * 8
            fatb = fata + 4
            dena = [jnp.zeros((16,), jnp.float32) for _ in range(4)]
            svna = [jnp.zeros((16,), jnp.float32) for _ in range(4)]
            denb = [jnp.zeros((16,), jnp.float32) for _ in range(4)]
            svnb = [jnp.zeros((16,), jnp.float32) for _ in range(4)]
            for k in range(K):
                col = lane3_32 + ((lane + k) & (K - 1))
                va = plsc.load_gather(buf2d, [fata, col])
                vb = plsc.load_gather(buf2d, [fatb, col])
                ta = va * c1
                tb = vb * c1
                ea = jnp.exp(jnp.maximum(ta, ta * 0.2))
                eb = jnp.exp(jnp.maximum(tb, tb * 0.2))
                dena[k % 4] = dena[k % 4] + ea
                svna[k % 4] = svna[k % 4] + va * ea
                denb[k % 4] = denb[k % 4] + eb
                svnb[k % 4] = svnb[k % 4] + vb * eb
            da = (dena[0] + dena[1]) + (dena[2] + dena[3])
            sa = (svna[0] + svna[1]) + (svna[2] + svna[3])
            db = (denb[0] + denb[1]) + (denb[2] + denb[3])
            sb = (svnb[0] + svnb[1]) + (svnb[2] + svnb[3])
            oa = (sa * wv) / (da + 1e-16) + bg
            ob = (sb * wv) / (db + 1e-16) + bg
            ra = jnp.where(oa > 0, oa, jnp.exp(oa) - 1.0)
            rb = jnp.where(ob > 0, ob, jnp.exp(ob) - 1.0)
            outbuf[pl.ds(j * 32, 16)] = ra * wr + br
            outbuf[pl.ds(j * 32 + 16, 16)] = rb * wr + br
            return carry

        lax.fori_loop(0, G_PER_SLAB // 2, pairgroup, 0)

    def start(i):
        s = wid + N_WORKERS * i
        b = i % 2
        idxr = idxs[b]
        base = SC_FAT0 + s * FAT_PER_SLAB
        for t in range(IDX_PAD // 16):
            idxr[pl.ds(t * 16, 16)] = base + t * 16 + lane
        for off, sz in CHUNKS:
            pltpu.async_copy(
                neigh_hbm.at[idxr.at[pl.ds(off, sz)]],
                bufs[b].at[pl.ds(off, sz), :],
                sems[b])

    def wait_in(i):
        b = i % 2
        idxr = idxs[b]
        for off, sz in CHUNKS:
            pltpu.make_async_copy(
                neigh_hbm.at[idxr.at[pl.ds(off, sz)]],
                bufs[b].at[pl.ds(off, sz), :],
                sems[b]).wait()

    start(0)
    for i in range(SLABS_PER_W):
        s = wid + N_WORKERS * i
        b = i % 2
        if i + 1 < SLABS_PER_W:
            start(i + 1)
        wait_in(i)
        compute_slab(bufs[b], outbufs[b])
        pltpu.async_copy(
            outbufs[b], out_hbm.at[pl.ds(s * SC_SLAB_ROWS, SC_SLAB_ROWS)],
            osems[b])
    for i in range(SLABS_PER_W):
        s = wid + N_WORKERS * i
        b = i % 2
        pltpu.make_async_copy(
            outbufs[b], out_hbm.at[pl.ds(s * SC_SLAB_ROWS, SC_SLAB_ROWS)],
            osems[b]).wait()


def _tc_body(x_ref, c1s, wvs, bgs, wrs, brs, o_ref):
    x = x_ref[...]
    c1 = c1s[0, 0]
    t = x * c1
    e = jnp.exp(jnp.maximum(t, t * 0.2))
    seg = (lax.broadcasted_iota(jnp.int32, (FAT_COLS, FAT), 0) // K
           == lax.broadcasted_iota(jnp.int32, (FAT_COLS, FAT), 1)
           ).astype(jnp.bfloat16)
    dn = (((1,), (0,)), ((), ()))

    def segsum(v):
        hi = v.astype(jnp.bfloat16)
        lo = (v - hi.astype(jnp.float32)).astype(jnp.bfloat16)
        return (lax.dot_general(hi, seg, dn,
                                preferred_element_type=jnp.float32)
                + lax.dot_general(lo, seg, dn,
                                  preferred_element_type=jnp.float32))

    den = segsum(e)
    sv = segsum(x * e)
    o = (sv * wvs[0, 0]) / (den + 1e-16) + bgs[0, 0]
    r = jnp.where(o > 0, o, jnp.exp(o) - 1.0)
    o_ref[...] = r * wrs[0, 0] + brs[0, 0]


def kernel(query_emb, entity_emb, neighbors, W, a_src, a_tgt, b_gat, W_rank, b_rank):
    n = neighbors.shape[0]
    neigh2d = neighbors.reshape(n * K // FAT_COLS, FAT_COLS)
    w0 = W[0, 0]
    c1 = w0 * a_src[0, 0, 0]
    consts = jnp.concatenate([
        jnp.full((16,), c1, jnp.float32),
        jnp.full((16,), w0, jnp.float32),
        jnp.full((16,), b_gat[0], jnp.float32),
        jnp.full((16,), W_rank[0, 0], jnp.float32),
        jnp.full((16,), b_rank[0], jnp.float32),
    ])

    mesh = plsc.VectorSubcoreMesh(core_axis_name="c", subcore_axis_name="s")
    sc_run = functools.partial(
        pl.kernel,
        mesh=mesh,
        compiler_params=pltpu.CompilerParams(needs_layout_passes=False),
        out_type=jax.ShapeDtypeStruct((SC_ROWS,), jnp.float32),
        scratch_types=[
            pltpu.VMEM((FAT_PER_SLAB, FAT_COLS), jnp.float32),
            pltpu.VMEM((FAT_PER_SLAB, FAT_COLS), jnp.float32),
            pltpu.VMEM((SC_SLAB_ROWS,), jnp.float32),
            pltpu.VMEM((SC_SLAB_ROWS,), jnp.float32),
            pltpu.VMEM((IDX_PAD,), jnp.int32),
            pltpu.VMEM((IDX_PAD,), jnp.int32),
            pltpu.VMEM((80,), jnp.float32),
            pltpu.SemaphoreType.DMA,
            pltpu.SemaphoreType.DMA,
            pltpu.SemaphoreType.DMA,
            pltpu.SemaphoreType.DMA,
        ],
    )(_sc_body)
    out_sc = sc_run(neigh2d, consts)

    s11 = lambda v: jnp.full((1, 1), v, jnp.float32)
    smem_spec = pl.BlockSpec(memory_space=pltpu.SMEM)
    out_tc = pl.pallas_call(
        _tc_body,
        grid=(TC_GRID,),
        in_specs=[
            pl.BlockSpec((TC_BLOCK_FAT, FAT_COLS), lambda i: (i, 0)),
            smem_spec, smem_spec, smem_spec, smem_spec, smem_spec,
        ],
        out_specs=pl.BlockSpec((TC_BLOCK_FAT, FAT), lambda i: (i, 0)),
        out_shape=jax.ShapeDtypeStruct((TC_FAT, FAT), jnp.float32),
        compiler_params=pltpu.CompilerParams(
            dimension_semantics=("parallel",)),
    )(neigh2d, s11(c1), s11(w0), s11(b_gat[0]), s11(W_rank[0, 0]),
      s11(b_rank[0]))

    out = jnp.concatenate([out_tc.reshape(TC_ROWS), out_sc])
    return out.reshape(n, 1)


# speedup vs baseline: 1.1109x; 1.0463x over previous
import functools
import jax
import jax.numpy as jnp
from jax import lax
from jax.experimental import pallas as pl
from jax.experimental.pallas import tpu as pltpu, tpu_sc as plsc

N_ROWS = 100000
K = 32
SLAB_ROWS = 800
G_PER_SLAB = SLAB_ROWS // 16
N_SLABS = N_ROWS // SLAB_ROWS
N_WORKERS = 32
MAX_SLABS_PER_W = (N_SLABS + N_WORKERS - 1) // N_WORKERS
FAT = 4
FAT_COLS = FAT * K
FAT_PER_SLAB = SLAB_ROWS // FAT
CHUNKS = ((0, 104), (104, 96))
IDX_PAD = 208


def _body(neigh_hbm, consts_hbm, out_hbm, buf0, buf1, outbuf0, outbuf1,
          idx0, idx1, cbuf, sem0a, sem0b, sem1a, sem1b, osem0, osem1):
    wid = lax.axis_index("s") * 2 + lax.axis_index("c")
    pltpu.sync_copy(consts_hbm, cbuf)
    c1 = cbuf[pl.ds(0, 16)]
    wv = cbuf[pl.ds(16, 16)]
    bg = cbuf[pl.ds(32, 16)]
    wr = cbuf[pl.ds(48, 16)]
    br = cbuf[pl.ds(64, 16)]
    lane = lax.iota(jnp.int32, 16)

    sems = ((sem0a, sem0b), (sem1a, sem1b))
    bufs = (buf0, buf1)
    osems = (osem0, osem1)
    outbufs = (outbuf0, outbuf1)
    idxs = (idx0, idx1)

    lane4 = lax.shift_right_logical(lane, 2)
    lane3_32 = (lane & 3) * K

    def compute_range(buf2d, outbuf, p0, p1):
        def pairgroup(j, carry):
            fata = lane4 + j * 8
            fatb = fata + 4
            dena = [jnp.zeros((16,), jnp.float32) for _ in range(4)]
            svna = [jnp.zeros((16,), jnp.float32) for _ in range(4)]
            denb = [jnp.zeros((16,), jnp.float32) for _ in range(4)]
            svnb = [jnp.zeros((16,), jnp.float32) for _ in range(4)]
            for k in range(K):
                col = lane3_32 + ((lane + k) & (K - 1))
                va = plsc.load_gather(buf2d, [fata, col])
                vb = plsc.load_gather(buf2d, [fatb, col])
                ta = va * c1
                tb = vb * c1
                ea = jnp.exp(jnp.maximum(ta, ta * 0.2))
                eb = jnp.exp(jnp.maximum(tb, tb * 0.2))
                dena[k % 4] = dena[k % 4] + ea
                svna[k % 4] = svna[k % 4] + va * ea
                denb[k % 4] = denb[k % 4] + eb
                svnb[k % 4] = svnb[k % 4] + vb * eb
            da = (dena[0] + dena[1]) + (dena[2] + dena[3])
            sa = (svna[0] + svna[1]) + (svna[2] + svna[3])
            db = (denb[0] + denb[1]) + (denb[2] + denb[3])
            sb = (svnb[0] + svnb[1]) + (svnb[2] + svnb[3])
            oa = (sa * wv) / (da + 1e-16) + bg
            ob = (sb * wv) / (db + 1e-16) + bg
            ra = jnp.where(oa > 0, oa, jnp.exp(oa) - 1.0)
            rb = jnp.where(ob > 0, ob, jnp.exp(ob) - 1.0)
            outbuf[pl.ds(j * 32, 16)] = ra * wr + br
            outbuf[pl.ds(j * 32 + 16, 16)] = rb * wr + br
            return carry

        lax.fori_loop(p0, p1, pairgroup, 0)

    def start(i):
        s = wid + N_WORKERS * i
        b = i % 2
        idxr = idxs[b]
        base = s * FAT_PER_SLAB
        for t in range(IDX_PAD // 16):
            idxr[pl.ds(t * 16, 16)] = base + t * 16 + lane
        for c, (off, sz) in enumerate(CHUNKS):
            pltpu.async_copy(
                neigh_hbm.at[idxr.at[pl.ds(off, sz)]],
                bufs[b].at[pl.ds(off, sz), :],
                sems[b][c])

    def consume(i):
        b = i % 2
        idxr = idxs[b]
        for c, (off, sz) in enumerate(CHUNKS):
            pltpu.make_async_copy(
                neigh_hbm.at[idxr.at[pl.ds(off, sz)]],
                bufs[b].at[pl.ds(off, sz), :],
                sems[b][c]).wait()
            compute_range(bufs[b], outbufs[b], off // 8, (off + sz) // 8)

    start(0)
    for i in range(MAX_SLABS_PER_W):
        s = wid + N_WORKERS * i
        b = i % 2
        if i + 1 < MAX_SLABS_PER_W:
            @pl.when(wid + N_WORKERS * (i + 1) < N_SLABS)
            def _():
                start(i + 1)

        @pl.when(s < N_SLABS)
        def _():
            if i >= 2:
                pltpu.make_async_copy(
                    outbufs[b],
                    out_hbm.at[pl.ds((s - 2 * N_WORKERS) * SLAB_ROWS,
                                     SLAB_ROWS)],
                    osems[b]).wait()
            consume(i)
            pltpu.async_copy(
                outbufs[b], out_hbm.at[pl.ds(s * SLAB_ROWS, SLAB_ROWS)],
                osems[b])

    for i in range(MAX_SLABS_PER_W):
        s = wid + N_WORKERS * i
        b = i % 2

        @pl.when(jnp.logical_and(s < N_SLABS, s + 2 * N_WORKERS >= N_SLABS))
        def _():
            pltpu.make_async_copy(
                outbufs[b], out_hbm.at[pl.ds(s * SLAB_ROWS, SLAB_ROWS)],
                osems[b]).wait()


def kernel(query_emb, entity_emb, neighbors, W, a_src, a_tgt, b_gat, W_rank, b_rank):
    n = neighbors.shape[0]
    neigh2d = neighbors.reshape(n * K // FAT_COLS, FAT_COLS)
    w0 = W[0, 0]
    consts = jnp.concatenate([
        jnp.full((16,), w0 * a_src[0, 0, 0], jnp.float32),
        jnp.full((16,), w0, jnp.float32),
        jnp.full((16,), b_gat[0], jnp.float32),
        jnp.full((16,), W_rank[0, 0], jnp.float32),
        jnp.full((16,), b_rank[0], jnp.float32),
    ])

    mesh = plsc.VectorSubcoreMesh(core_axis_name="c", subcore_axis_name="s")
    run = functools.partial(
        pl.kernel,
        mesh=mesh,
        compiler_params=pltpu.CompilerParams(needs_layout_passes=False),
        out_type=jax.ShapeDtypeStruct((n,), jnp.float32),
        scratch_types=[
            pltpu.VMEM((FAT_PER_SLAB, FAT_COLS), jnp.float32),
            pltpu.VMEM((FAT_PER_SLAB, FAT_COLS), jnp.float32),
            pltpu.VMEM((SLAB_ROWS,), jnp.float32),
            pltpu.VMEM((SLAB_ROWS,), jnp.float32),
            pltpu.VMEM((IDX_PAD,), jnp.int32),
            pltpu.VMEM((IDX_PAD,), jnp.int32),
            pltpu.VMEM((80,), jnp.float32),
        ] + [pltpu.SemaphoreType.DMA] * 6,
    )(_body)
    out = run(neigh2d, consts)
    return out.reshape(n, 1)
